# baseline (speedup 1.0000x reference)
import functools

import jax
import jax.numpy as jnp
from jax import lax
from jax.experimental import pallas as pl
from jax.experimental.pallas import tpu as pltpu
from jax.experimental.pallas import tpu_sc as plsc

BATCH = 16384
EMBED = 32
LANES = 16

_info = plsc.get_sparse_core_info()
_NC = _info.num_cores
_NS = _info.num_subcores
NW = _NC * _NS
BPW = BATCH // NW
NCHUNK = 4
CH = BPW // NCHUNK
NGROUP = BPW // LANES


@functools.partial(
    pl.kernel,
    mesh=plsc.VectorSubcoreMesh(core_axis_name="c", subcore_axis_name="s"),
    out_type=jax.ShapeDtypeStruct((BATCH,), jnp.float32),
    compiler_params=pltpu.CompilerParams(
        needs_layout_passes=False, use_tc_tiling_on_sc=False),
    scratch_types=[
        pltpu.VMEM((NCHUNK, CH), jnp.int32),
        pltpu.VMEM((NCHUNK, CH), jnp.int32),
        pltpu.VMEM((BPW, EMBED), jnp.float32),
        pltpu.VMEM((BPW, EMBED), jnp.float32),
        pltpu.VMEM((BPW,), jnp.float32),
        pltpu.SemaphoreType.DMA,
        pltpu.SemaphoreType.DMA,
    ],
)
def _mf_kernel(user_hbm, item_hbm, ut_hbm, it_hbm, out_hbm,
               idx_u, idx_i, rows_u, rows_i, out_v, sem_u, sem_i):
    wid = lax.axis_index("s") * _NC + lax.axis_index("c")
    base = wid * BPW

    for j in range(NCHUNK):
        pltpu.sync_copy(user_hbm.at[pl.ds(base + j * CH, CH)], idx_u.at[j])
        pltpu.sync_copy(item_hbm.at[pl.ds(base + j * CH, CH)], idx_i.at[j])

    copies = []
    for j in range(NCHUNK):
        copies.append(pltpu.async_copy(
            ut_hbm.at[idx_u.at[j]], rows_u.at[pl.ds(j * CH, CH)], sem_u))
        copies.append(pltpu.async_copy(
            it_hbm.at[idx_i.at[j]], rows_i.at[pl.ds(j * CH, CH)], sem_i))
    for c in copies:
        c.wait()

    iota = lax.iota(jnp.int32, LANES)

    def group_body(g, carry):
        rows = g * LANES + iota
        acc = jnp.zeros((LANES,), jnp.float32)
        for d in range(EMBED):
            col = jnp.full((LANES,), d, jnp.int32)
            u = plsc.load_gather(rows_u, [rows, col])
            v = plsc.load_gather(rows_i, [rows, col])
            acc = acc + u * v
        out_v[pl.ds(g * LANES, LANES)] = acc
        return carry

    lax.fori_loop(0, NGROUP, group_body, 0)

    pltpu.sync_copy(out_v, out_hbm.at[pl.ds(base, BPW)])


def kernel(user, item, user_table, item_table):
    return _mf_kernel(user, item, user_table, item_table)


# speedup vs baseline: 1.0175x; 1.0175x over previous
import functools

import jax
import jax.numpy as jnp
from jax import lax
from jax.experimental import pallas as pl
from jax.experimental.pallas import tpu as pltpu
from jax.experimental.pallas import tpu_sc as plsc

BATCH = 16384
EMBED = 32
LANES = 16
IDXW = 128

_info = plsc.get_sparse_core_info()
_NC = _info.num_cores
_NS = _info.num_subcores
NW = _NC * _NS
BPW = BATCH // NW
NCHUNK = BPW // IDXW
NGROUP = BPW // LANES


@functools.partial(
    pl.kernel,
    mesh=plsc.VectorSubcoreMesh(core_axis_name="c", subcore_axis_name="s"),
    out_type=jax.ShapeDtypeStruct((BATCH,), jnp.float32),
    compiler_params=pltpu.CompilerParams(
        needs_layout_passes=False, use_tc_tiling_on_sc=False),
    scratch_types=[
        pltpu.VMEM((NCHUNK, IDXW), jnp.int32),
        pltpu.VMEM((NCHUNK, IDXW), jnp.int32),
        pltpu.VMEM((BPW, EMBED), jnp.float32),
        pltpu.VMEM((BPW, EMBED), jnp.float32),
        pltpu.VMEM((LANES, LANES), jnp.float32),
        pltpu.VMEM((BPW,), jnp.float32),
        pltpu.SemaphoreType.DMA,
        pltpu.SemaphoreType.DMA,
    ],
)
def _mf_kernel(user_hbm, item_hbm, ut_hbm, it_hbm, out_hbm,
               idx_u, idx_i, rows_u, rows_i, tbuf, out_v, sem_u, sem_i):
    wid = lax.axis_index("s") * _NC + lax.axis_index("c")

    pltpu.sync_copy(user_hbm.at[pl.ds(wid * NCHUNK, NCHUNK)], idx_u)
    pltpu.sync_copy(item_hbm.at[pl.ds(wid * NCHUNK, NCHUNK)], idx_i)

    copies = []
    for j in range(NCHUNK):
        copies.append(pltpu.async_copy(
            ut_hbm.at[idx_u.at[j]], rows_u.at[pl.ds(j * IDXW, IDXW)], sem_u))
        copies.append(pltpu.async_copy(
            it_hbm.at[idx_i.at[j]], rows_i.at[pl.ds(j * IDXW, IDXW)], sem_i))
    for c in copies:
        c.wait()

    row_iota = lax.iota(jnp.int32, LANES)

    def group_body(g, carry):
        base = g * LANES
        for i in range(LANES):
            b = base + i
            p = (rows_u[b, pl.ds(0, LANES)] * rows_i[b, pl.ds(0, LANES)]
                 + rows_u[b, pl.ds(LANES, LANES)] * rows_i[b, pl.ds(LANES, LANES)])
            tbuf[i, pl.ds(0, LANES)] = p
        acc = plsc.load_gather(tbuf, [row_iota, jnp.zeros((LANES,), jnp.int32)])
        for d in range(1, LANES):
            acc = acc + plsc.load_gather(
                tbuf, [row_iota, jnp.full((LANES,), d, jnp.int32)])
        out_v[pl.ds(base, LANES)] = acc
        return carry

    lax.fori_loop(0, NGROUP, group_body, 0)

    pltpu.sync_copy(out_v, out_hbm.at[pl.ds(wid * BPW, BPW)])


def kernel(user, item, user_table, item_table):
    return _mf_kernel(user.reshape(NW * NCHUNK, IDXW),
                      item.reshape(NW * NCHUNK, IDXW),
                      user_table, item_table)
